# baseline (speedup 1.0000x reference)
import functools

import numpy as np
import jax
import jax.numpy as jnp
from jax import lax
from jax.experimental import pallas as pl
from jax.experimental.pallas import tpu as pltpu
from jax.experimental.pallas import tpu_sc as plsc

_B = 1024
_N = 4096


def _make_perm() -> np.ndarray:
    np.random.seed(42)
    ind = np.arange(_N)
    np.random.shuffle(ind)
    return ind.astype(np.int32)


_PERM = _make_perm()

_NC = 2
_NS = 16
_L = 16
_NW = _NC * _NS
_ROWS_PER_W = _B // _NW
_C = 4
_H = _C // 2
_NCHUNK = _ROWS_PER_W // _C
_SLOTS_IN = 4
_SLOTS_OUT = 3
_UNROLL = 2


@functools.cache
def _build():
    mesh = plsc.VectorSubcoreMesh(core_axis_name="c", subcore_axis_name="s")

    @functools.partial(
        pl.kernel,
        mesh=mesh,
        out_type=jax.ShapeDtypeStruct((_B, _N), jnp.float32),
        scratch_types=[
            pltpu.VMEM((_N,), jnp.int32),
            pltpu.VMEM((_SLOTS_IN, _C, _N), jnp.float32),
            pltpu.VMEM((_SLOTS_OUT, _C, _N), jnp.float32),
            pltpu.SemaphoreType.DMA((_SLOTS_IN,)),
            pltpu.SemaphoreType.DMA((_SLOTS_OUT,)),
            pltpu.SemaphoreType.DMA,
        ],
        compiler_params=pltpu.CompilerParams(needs_layout_passes=False),
    )
    def shuffle_sc(x_hbm, perm_hbm, out_hbm, idx_v, in_v, out_v,
                   in_sems, out_sems, perm_sem):
        wid = lax.axis_index("s") * _NC + lax.axis_index("c")
        row0 = wid * _ROWS_PER_W

        def in_copy(g, slot):
            return pltpu.make_async_copy(
                x_hbm.at[pl.ds(row0 + g * _C, _C)], in_v.at[slot],
                in_sems.at[slot])

        def out_half(g, slot, h):
            return pltpu.make_async_copy(
                out_v.at[slot, pl.ds(h * _H, _H)],
                out_hbm.at[pl.ds(row0 + g * _C + h * _H, _H)],
                out_sems.at[slot])

        perm_dma = pltpu.make_async_copy(perm_hbm, idx_v, perm_sem)
        perm_dma.start()
        for s in range(_SLOTS_IN):
            in_copy(s, s).start()
        perm_dma.wait()

        def chunk_body(g, carry):
            bi = g % _SLOTS_IN
            b = g % _SLOTS_OUT
            in_copy(g, bi).wait()

            @pl.when(g >= _SLOTS_OUT)
            def _():
                out_half(g - _SLOTS_OUT, b, 0).wait()
                out_half(g - _SLOTS_OUT, b, 1).wait()

            bi_idx = jnp.full((_L,), bi, jnp.int32)

            for h in range(2):
                @plsc.parallel_loop(0, _N // _L, unroll=_UNROLL)
                def _gather(jb, h=h):
                    col = jb * _L
                    idx = idx_v[pl.ds(col, _L)]
                    for r in range(h * _H, (h + 1) * _H):
                        row_idx = jnp.full((_L,), r, jnp.int32)
                        out_v[b, r, pl.ds(col, _L)] = plsc.load_gather(
                            in_v, [bi_idx, row_idx, idx])

                out_half(g, b, h).start()

            @pl.when(g + _SLOTS_IN < _NCHUNK)
            def _():
                in_copy(g + _SLOTS_IN, bi).start()

            return carry

        lax.fori_loop(0, _NCHUNK, chunk_body, 0)
        for g in range(_NCHUNK - _SLOTS_OUT, _NCHUNK):
            out_half(g, g % _SLOTS_OUT, 0).wait()
            out_half(g, g % _SLOTS_OUT, 1).wait()

    return shuffle_sc


@functools.cache
def _perm_on_device():
    return jax.device_put(jnp.asarray(_PERM))


def kernel(x):
    return _build()(x, _perm_on_device())


# speedup vs baseline: 1.1346x; 1.1346x over previous
import functools

import numpy as np
import jax
import jax.numpy as jnp
from jax import lax
from jax.experimental import pallas as pl
from jax.experimental.pallas import tpu as pltpu
from jax.experimental.pallas import tpu_sc as plsc

_B = 1024
_N = 4096


def _make_perm() -> np.ndarray:
    np.random.seed(42)
    ind = np.arange(_N)
    np.random.shuffle(ind)
    return ind.astype(np.int32)


_PERM = _make_perm()

_NC = 2
_NS = 16
_L = 16
_NW = _NC * _NS
_ROWS_PER_W = _B // _NW
_C = 4
_NCHUNK = _ROWS_PER_W // _C
_SLOTS_IN = 4
_SLOTS_OUT = 3
_UNROLL = 2


@functools.cache
def _build():
    mesh = plsc.VectorSubcoreMesh(core_axis_name="c", subcore_axis_name="s")

    @functools.partial(
        pl.kernel,
        mesh=mesh,
        out_type=jax.ShapeDtypeStruct((_B, _N), jnp.float32),
        scratch_types=[
            pltpu.VMEM((_N,), jnp.int32),
            pltpu.VMEM((_SLOTS_IN, _C, _N), jnp.float32),
            pltpu.VMEM((_SLOTS_OUT, _C, _N), jnp.float32),
            pltpu.SemaphoreType.DMA((_SLOTS_IN,)),
            pltpu.SemaphoreType.DMA((_SLOTS_OUT,)),
            pltpu.SemaphoreType.DMA,
        ],
        compiler_params=pltpu.CompilerParams(needs_layout_passes=False),
    )
    def shuffle_sc(x_hbm, perm_hbm, out_hbm, idx_v, in_v, out_v,
                   in_sems, out_sems, perm_sem):
        wid = lax.axis_index("s") * _NC + lax.axis_index("c")
        row0 = wid * _ROWS_PER_W

        def in_copy(g, slot):
            return pltpu.make_async_copy(
                x_hbm.at[pl.ds(row0 + g * _C, _C)], in_v.at[slot],
                in_sems.at[slot])

        def out_copy(g, slot):
            return pltpu.make_async_copy(
                out_v.at[slot], out_hbm.at[pl.ds(row0 + g * _C, _C)],
                out_sems.at[slot])

        perm_dma = pltpu.make_async_copy(perm_hbm, idx_v, perm_sem)
        perm_dma.start()
        for s in range(_SLOTS_IN):
            in_copy(s, s).start()
        perm_dma.wait()

        def chunk_body(g, carry):
            bi = g % _SLOTS_IN
            b = g % _SLOTS_OUT
            in_copy(g, bi).wait()

            @pl.when(g >= _SLOTS_OUT)
            def _():
                out_copy(g - _SLOTS_OUT, b).wait()

            bi_idx = jnp.full((_L,), bi, jnp.int32)

            @plsc.parallel_loop(0, _N // _L, unroll=_UNROLL)
            def _gather(jb):
                col = jb * _L
                idx = idx_v[pl.ds(col, _L)]
                for r in range(_C):
                    row_idx = jnp.full((_L,), r, jnp.int32)
                    out_v[b, r, pl.ds(col, _L)] = plsc.load_gather(
                        in_v, [bi_idx, row_idx, idx])

            out_copy(g, b).start()

            @pl.when(g + _SLOTS_IN < _NCHUNK)
            def _():
                in_copy(g + _SLOTS_IN, bi).start()

            return carry

        lax.fori_loop(0, _NCHUNK, chunk_body, 0)
        for g in range(_NCHUNK - _SLOTS_OUT, _NCHUNK):
            out_copy(g, g % _SLOTS_OUT).wait()

    return shuffle_sc


@functools.cache
def _perm_on_device():
    return jax.device_put(jnp.asarray(_PERM))


def kernel(x):
    return _build()(x, _perm_on_device())
